# baseline (speedup 1.0000x reference)
import functools

import numpy as np
import jax
import jax.numpy as jnp
from jax import lax
from jax.experimental import pallas as pl
from jax.experimental.pallas import tpu as pltpu
from jax.experimental.pallas import tpu_sc as plsc

_NUM_FIELDS = 26
_FIELD_SIZE = 100000
_BATCH = 4096
_OUT_F = 64

_NW = 32
_CHUNK = 128
_TOTAL = _BATCH * _NUM_FIELDS
_PER_W = _TOTAL // _NW
_K = _PER_W // _CHUNK

_OFF2D = ((np.arange(_PER_W) % _NUM_FIELDS) * _FIELD_SIZE).astype(np.int32)
_OFF2D = _OFF2D.reshape(_K, _CHUNK)

_mesh = plsc.VectorSubcoreMesh(core_axis_name="c", subcore_axis_name="s")


@functools.partial(
    pl.kernel,
    mesh=_mesh,
    out_type=jax.ShapeDtypeStruct((_TOTAL, _OUT_F), jnp.float32),
    scratch_types=[
        pltpu.VMEM((_K, _CHUNK), jnp.int32),
        pltpu.VMEM((_K, _CHUNK), jnp.int32),
        pltpu.VMEM((_CHUNK, _OUT_F), jnp.float32),
        pltpu.SemaphoreType.DMA,
    ],
    compiler_params=pltpu.CompilerParams(use_tc_tiling_on_sc=False),
)
def _sc_gather(x_hbm, off_hbm, table_hbm, out_hbm, idx_v, off_v, rows_v, sem):
    wid = lax.axis_index("s") * 2 + lax.axis_index("c")
    pltpu.sync_copy(x_hbm.at[wid], idx_v)
    pltpu.sync_copy(off_hbm, off_v)

    def add_body(r, carry):
        for c in range(_CHUNK // 16):
            sl = pl.ds(c * 16, 16)
            idx_v[r, sl] = idx_v[r, sl] + off_v[r, sl]
        return carry

    lax.fori_loop(0, _K, add_body, None)

    base = wid * _PER_W

    def gather_body(j, carry):
        pltpu.async_copy(table_hbm.at[idx_v.at[j]], rows_v, sem).wait()
        pltpu.sync_copy(rows_v, out_hbm.at[pl.ds(base + j * _CHUNK, _CHUNK)])
        return carry

    lax.fori_loop(0, _K, gather_body, None)


def kernel(x, table):
    x2 = x.reshape(_NW, _K, _CHUNK)
    off2 = jnp.asarray(_OFF2D)
    out = _sc_gather(x2, off2, table)
    return out.reshape(_BATCH, _NUM_FIELDS, _OUT_F)


# speedup vs baseline: 1.0133x; 1.0133x over previous
import functools

import numpy as np
import jax
import jax.numpy as jnp
from jax import lax
from jax.experimental import pallas as pl
from jax.experimental.pallas import tpu as pltpu
from jax.experimental.pallas import tpu_sc as plsc

_NUM_FIELDS = 26
_FIELD_SIZE = 100000
_BATCH = 4096
_OUT_F = 64

_NW = 32
_TOTAL = _BATCH * _NUM_FIELDS
_PER_W = _TOTAL // _NW
_K = 4
_CHUNK = _PER_W // _K
_NBUF = 2

_OFF2D = ((np.arange(_PER_W) % _NUM_FIELDS) * _FIELD_SIZE).astype(np.int32)
_OFF2D = _OFF2D.reshape(_K, _CHUNK)

_mesh = plsc.VectorSubcoreMesh(core_axis_name="c", subcore_axis_name="s")


@functools.partial(
    pl.kernel,
    mesh=_mesh,
    out_type=jax.ShapeDtypeStruct((_TOTAL, _OUT_F), jnp.float32),
    scratch_types=[
        pltpu.VMEM((_K, _CHUNK), jnp.int32),
        pltpu.VMEM((_K, _CHUNK), jnp.int32),
        pltpu.VMEM((_NBUF, _CHUNK, _OUT_F), jnp.float32),
        pltpu.SemaphoreType.DMA((_NBUF,)),
        pltpu.SemaphoreType.DMA((_NBUF,)),
    ],
    compiler_params=pltpu.CompilerParams(use_tc_tiling_on_sc=False),
)
def _sc_gather(x_hbm, off_hbm, table_hbm, out_hbm, idx_v, off_v, rows_v, gsem, osem):
    wid = lax.axis_index("s") * 2 + lax.axis_index("c")
    pltpu.sync_copy(x_hbm.at[wid], idx_v)
    pltpu.sync_copy(off_hbm, off_v)

    def add_body(r, carry):
        for c in range(_CHUNK // 16):
            sl = pl.ds(c * 16, 16)
            idx_v[r, sl] = idx_v[r, sl] + off_v[r, sl]
        return carry

    lax.fori_loop(0, _K, add_body, None)

    base = wid * _PER_W

    def start_gather(m, b):
        return pltpu.async_copy(table_hbm.at[idx_v.at[m]], rows_v.at[b], gsem.at[b])

    def start_out(m, b):
        return pltpu.async_copy(
            rows_v.at[b], out_hbm.at[pl.ds(base + m * _CHUNK, _CHUNK)], osem.at[b]
        )

    gd = [None] * _NBUF
    od = [None] * _NBUF
    for b in range(_NBUF):
        gd[b] = start_gather(b, b)
    for m in range(_K):
        b = m % _NBUF
        gd[b].wait()
        od[b] = start_out(m, b)
        if m + _NBUF < _K:
            od[b].wait()
            gd[b] = start_gather(m + _NBUF, b)
    for b in range(_NBUF):
        od[b].wait()


def kernel(x, table):
    x2 = x.reshape(_NW, _K, _CHUNK)
    off2 = jnp.asarray(_OFF2D)
    out = _sc_gather(x2, off2, table)
    return out.reshape(_BATCH, _NUM_FIELDS, _OUT_F)


# speedup vs baseline: 1.3261x; 1.3086x over previous
import functools

import numpy as np
import jax
import jax.numpy as jnp
from jax import lax
from jax.experimental import pallas as pl
from jax.experimental.pallas import tpu as pltpu
from jax.experimental.pallas import tpu_sc as plsc

_NUM_FIELDS = 26
_FIELD_SIZE = 100000
_BATCH = 4096
_OUT_F = 64

_NW = 32
_TOTAL = _BATCH * _NUM_FIELDS
_PER_W = _TOTAL // _NW
_C = 32
_NCHUNK = _PER_W // _C

_OFF = ((np.arange(_PER_W) % _NUM_FIELDS) * _FIELD_SIZE).astype(np.int32)

_mesh = plsc.VectorSubcoreMesh(core_axis_name="c", subcore_axis_name="s")


@functools.partial(
    pl.kernel,
    mesh=_mesh,
    out_type=jax.ShapeDtypeStruct((_TOTAL, _OUT_F), jnp.float32),
    scratch_types=[
        pltpu.VMEM((_PER_W,), jnp.int32),
        pltpu.VMEM((_PER_W,), jnp.int32),
        pltpu.VMEM((_C, 8, _OUT_F), jnp.float32),
        pltpu.VMEM((_C, _OUT_F), jnp.float32),
        pltpu.SemaphoreType.DMA,
    ],
)
def _sc_gather(x_hbm, off_hbm, table_hbm, out_hbm,
               idx_v, off_v, tiles_v, stage_v, sem):
    wid = lax.axis_index("s") * 2 + lax.axis_index("c")
    base = wid * _PER_W
    pltpu.sync_copy(x_hbm.at[pl.ds(base, _PER_W)], idx_v)
    pltpu.sync_copy(off_hbm, off_v)

    def prep_body(r, carry):
        sl = pl.ds(r * 16, 16)
        idx_v[sl] = idx_v[sl] + off_v[sl]
        return carry

    lax.fori_loop(0, _PER_W // 16, prep_body, None)

    def chunk_body(kl, carry):
        descs = []
        svals = []
        for v16 in range(_C // 16):
            vec = idx_v[pl.ds(kl * _C + v16 * 16, 16)]
            for l in range(16):
                svals.append(jax.lax.index_in_dim(vec, l, 0, keepdims=False))
        for n in range(_C):
            g8 = pl.multiple_of(lax.bitwise_and(svals[n], -8), 8)
            descs.append(
                pltpu.async_copy(table_hbm.at[pl.ds(g8, 8)], tiles_v.at[n], sem)
            )
        for d in descs:
            d.wait()
        for n in range(_C):
            i = lax.bitwise_and(svals[n], 7)
            for c in range(_OUT_F // 16):
                sl = pl.ds(c * 16, 16)
                stage_v[n, sl] = tiles_v[n, i, sl]
        pltpu.sync_copy(stage_v, out_hbm.at[pl.ds(base + kl * _C, _C)])
        return carry

    lax.fori_loop(0, _NCHUNK, chunk_body, None)


def kernel(x, table):
    xf = x.reshape(_TOTAL)
    off = jnp.asarray(_OFF)
    out = _sc_gather(xf, off, table)
    return out.reshape(_BATCH, _NUM_FIELDS, _OUT_F)


# speedup vs baseline: 1.4216x; 1.0720x over previous
import functools

import numpy as np
import jax
import jax.numpy as jnp
from jax import lax
from jax.experimental import pallas as pl
from jax.experimental.pallas import tpu as pltpu
from jax.experimental.pallas import tpu_sc as plsc

_NUM_FIELDS = 26
_FIELD_SIZE = 100000
_BATCH = 4096
_OUT_F = 64

_NW = 32
_TOTAL = _BATCH * _NUM_FIELDS
_PER_W = _TOTAL // _NW
_C = 32
_NCHUNK = _PER_W // _C

_OFF = ((np.arange(_PER_W) % _NUM_FIELDS) * _FIELD_SIZE).astype(np.int32)

_mesh = plsc.VectorSubcoreMesh(core_axis_name="c", subcore_axis_name="s")


@functools.partial(
    pl.kernel,
    mesh=_mesh,
    out_type=jax.ShapeDtypeStruct((_TOTAL, _OUT_F), jnp.float32),
    scratch_types=[
        pltpu.VMEM((_PER_W,), jnp.int32),
        pltpu.VMEM((_PER_W,), jnp.int32),
        pltpu.VMEM((2, _C, 8, _OUT_F), jnp.float32),
        pltpu.VMEM((_C, _OUT_F), jnp.float32),
        pltpu.SemaphoreType.DMA((2,)),
    ],
)
def _sc_gather(x_hbm, off_hbm, drain_hbm, table_hbm, out_hbm,
               idx_v, off_v, tiles_v, stage_v, gsem):
    wid = lax.axis_index("s") * 2 + lax.axis_index("c")
    base = wid * _PER_W
    pltpu.sync_copy(x_hbm.at[pl.ds(base, _PER_W)], idx_v)
    pltpu.sync_copy(off_hbm, off_v)

    def prep_body(r, carry):
        sl = pl.ds(r * 16, 16)
        idx_v[sl] = idx_v[sl] + off_v[sl]
        return carry

    lax.fori_loop(0, _PER_W // 16, prep_body, None)

    def issue_round(k, b):
        svals = []
        for v16 in range(_C // 16):
            vec = idx_v[pl.ds(k * _C + v16 * 16, 16)]
            for l in range(16):
                svals.append(jax.lax.index_in_dim(vec, l, 0, keepdims=False))
        for n in range(_C):
            g8 = pl.multiple_of(lax.bitwise_and(svals[n], -8), 8)
            pltpu.async_copy(
                table_hbm.at[pl.ds(g8, 8)], tiles_v.at[b].at[n], gsem.at[b]
            )
        return svals

    issue_round(0, 0)

    def round_body(k, carry):
        b = lax.rem(k, 2)

        for bb in range(2):
            @pl.when(jnp.logical_and(b == bb, k + 1 < _NCHUNK))
            def _():
                issue_round(k + 1, 1 - bb)

        for bb in range(2):
            @pl.when(b == bb)
            def _():
                pltpu.make_async_copy(
                    drain_hbm, tiles_v.at[bb], gsem.at[bb]
                ).wait()
                svals = []
                for v16 in range(_C // 16):
                    vec = idx_v[pl.ds(k * _C + v16 * 16, 16)]
                    for l in range(16):
                        svals.append(
                            jax.lax.index_in_dim(vec, l, 0, keepdims=False)
                        )
                for n in range(_C):
                    i = lax.bitwise_and(svals[n], 7)
                    for c in range(_OUT_F // 16):
                        sl = pl.ds(c * 16, 16)
                        stage_v[n, sl] = tiles_v[bb, n, i, sl]
        pltpu.sync_copy(stage_v, out_hbm.at[pl.ds(base + k * _C, _C)])
        return carry

    lax.fori_loop(0, _NCHUNK, round_body, None)


def kernel(x, table):
    xf = x.reshape(_TOTAL)
    off = jnp.asarray(_OFF)
    drain = jnp.zeros((_C, 8, _OUT_F), jnp.float32)
    out = _sc_gather(xf, off, drain, table)
    return out.reshape(_BATCH, _NUM_FIELDS, _OUT_F)


# speedup vs baseline: 1.8950x; 1.3330x over previous
import functools

import numpy as np
import jax
import jax.numpy as jnp
from jax import lax
from jax.experimental import pallas as pl
from jax.experimental.pallas import tpu as pltpu
from jax.experimental.pallas import tpu_sc as plsc

_NUM_FIELDS = 26
_FIELD_SIZE = 100000
_BATCH = 4096
_OUT_F = 64

_NW = 32
_TOTAL = _BATCH * _NUM_FIELDS
_PER_W = _TOTAL // _NW
_R = 8
_NROUND = _PER_W // _R
_NP = 8
_TAILS = 2600000 - 64
_TAILP = _TAILS >> 7

_OFF = ((np.arange(_TOTAL) % _NUM_FIELDS) * _FIELD_SIZE).astype(np.int32)

_mesh = plsc.VectorSubcoreMesh(core_axis_name="c", subcore_axis_name="s")


@functools.partial(
    pl.kernel,
    mesh=_mesh,
    out_type=jax.ShapeDtypeStruct((_TOTAL * _OUT_F,), jnp.float32),
    scratch_types=[
        pltpu.VMEM((_PER_W + 16,), jnp.int32),
        pltpu.VMEM((_PER_W + 16,), jnp.int32),
        pltpu.VMEM((_NP * _OUT_F, 128), jnp.float32),
        pltpu.VMEM((_R * _OUT_F,), jnp.float32),
        pltpu.VMEM((_OUT_F, 64), jnp.float32),
        pltpu.SemaphoreType.DMA,
        pltpu.SemaphoreType.DMA,
    ],
    compiler_params=pltpu.CompilerParams(needs_layout_passes=False),
)
def _sc_gather(sk_hbm, sv_hbm, rowdrain_hbm, tabt_hbm, tailt_hbm, out_hbm,
               idx_v, pos_v, pbuf, rowstage, tailbuf, psem, osem):
    wid = lax.axis_index("s") * 2 + lax.axis_index("c")
    base = wid * _PER_W
    pltpu.sync_copy(sk_hbm.at[pl.ds(base, _PER_W)], idx_v.at[pl.ds(0, _PER_W)])
    pltpu.sync_copy(sv_hbm.at[pl.ds(base, _PER_W)], pos_v.at[pl.ds(0, _PER_W)])
    pltpu.sync_copy(tailt_hbm, tailbuf)

    iota16 = lax.iota(jnp.int32, 16)

    def drain_panel():
        pltpu.make_async_copy(
            tabt_hbm.at[:, pl.ds(0, 128)], pbuf.at[pl.ds(0, _OUT_F)], psem
        ).wait()

    def drain_row(n):
        pltpu.make_async_copy(
            rowdrain_hbm, rowstage.at[pl.ds(n * _OUT_F, _OUT_F)], osem
        ).wait()

    def issue_block(start, p_run, ic_run):
        vec = idx_v[pl.ds(start, 16)]
        for n in range(_R):
            s = jax.lax.index_in_dim(vec, n, 0, keepdims=False)
            pn = lax.shift_right_logical(s, 7)
            newp = jnp.logical_and(pn > p_run, pn < _TAILP)

            @pl.when(newp)
            def _(pn=pn, ic=ic_run):
                off = pl.multiple_of(pn * 128, 128)
                slot = lax.bitwise_and(ic, _NP - 1)
                pltpu.async_copy(
                    tabt_hbm.at[:, pl.ds(off, 128)],
                    pbuf.at[pl.ds(slot * _OUT_F, _OUT_F)],
                    psem,
                )

            ic_run = ic_run + newp.astype(jnp.int32)
            p_run = lax.max(p_run, pn)
        return p_run, ic_run

    def extract_block(start, p_run, ic_run):
        vec_i = idx_v[pl.ds(start, 16)]
        vec_p = pos_v[pl.ds(start, 16)]
        for n in range(_R):
            s = jax.lax.index_in_dim(vec_i, n, 0, keepdims=False)
            opos = jax.lax.index_in_dim(vec_p, n, 0, keepdims=False)
            pn = lax.shift_right_logical(s, 7)
            col = lax.bitwise_and(s, 127)
            newp = jnp.logical_and(pn > p_run, pn < _TAILP)
            ic_run = ic_run + newp.astype(jnp.int32)
            p_run = lax.max(p_run, pn)
            slot = lax.bitwise_and(ic_run - 1, _NP - 1)
            is_tail = s >= _TAILS

            @pl.when(jnp.logical_not(is_tail))
            def _(slot=slot, col=col, n=n):
                rbase = jnp.full((16,), slot * _OUT_F, jnp.int32) + iota16
                cvec = jnp.full((16,), col, jnp.int32)
                for c16 in range(_OUT_F // 16):
                    vals = plsc.load_gather(pbuf, [rbase + c16 * 16, cvec])
                    rowstage[pl.ds(n * _OUT_F + c16 * 16, 16)] = vals

            @pl.when(is_tail)
            def _(s=s, n=n):
                ct = jnp.full((16,), s - _TAILS, jnp.int32)
                for c16 in range(_OUT_F // 16):
                    vals = plsc.load_gather(tailbuf, [iota16 + c16 * 16, ct])
                    rowstage[pl.ds(n * _OUT_F + c16 * 16, 16)] = vals
            pltpu.async_copy(
                rowstage.at[pl.ds(n * _OUT_F, _OUT_F)],
                out_hbm.at[pl.ds(opos * _OUT_F, _OUT_F)], osem
            )

    def round_body(r, carry):
        p1, ic1, p2, ic2, dc = carry
        for j in range(_R):
            @pl.when(dc + j < ic1)
            def _():
                drain_panel()

        @pl.when(r > 1)
        def _():
            for n in range(_R):
                drain_row(n)

        @pl.when(r > 0)
        def _():
            extract_block((r - 1) * _R, p2, ic2)

        p1n, ic1n = issue_block(r * _R, p1, ic1)
        return p1n, ic1n, p1, ic1, ic1

    mone = jnp.int32(-1)
    zero = jnp.int32(0)
    p1, ic1, p2, ic2, dc = lax.fori_loop(
        0, _NROUND, round_body, (mone, zero, mone, zero, zero)
    )

    for j in range(_R):
        @pl.when(dc + j < ic1)
        def _():
            drain_panel()
    for n in range(_R):
        drain_row(n)
    extract_block((_NROUND - 1) * _R, p2, ic2)
    for n in range(_R):
        drain_row(n)


def kernel(x, table):
    xf = x.reshape(_TOTAL)
    idx = xf + jnp.asarray(_OFF)
    pos = jnp.arange(_TOTAL, dtype=jnp.int32)
    sk, sv = lax.sort((idx, pos), num_keys=1)
    tabt = table.T
    tailt = table[_TAILS:, :].T
    rowdrain = jnp.zeros((_OUT_F,), jnp.float32)
    out = _sc_gather(sk, sv, rowdrain, tabt, tailt)
    return out.reshape(_BATCH, _NUM_FIELDS, _OUT_F)
